# baseline (speedup 1.0000x reference)
import functools

import jax
import jax.numpy as jnp
from jax import lax
from jax.experimental import pallas as pl
from jax.experimental.pallas import tpu as pltpu
from jax.experimental.pallas import tpu_sc as plsc

N = 10000
E = 320000
D = 128
H = 64
L = 4
G = 64

CH = 128
GRP = 8
EPW = 10240
E_PAD = 32 * EPW
NCH = EPW // CH
NG = NCH // GRP
NP = 10112
HALF = NP // 2
AGR = 5120
DUMMY = HALF
STRIPE = AGR // 16


def _sc_half(h128, src3d, dst3d, zrows):
  mesh = plsc.VectorSubcoreMesh(core_axis_name="c", subcore_axis_name="s")

  @functools.partial(
      pl.kernel,
      out_type=[
          jax.ShapeDtypeStruct((AGR, 128), jnp.float32),
          jax.ShapeDtypeStruct((AGR, 128), jnp.float32),
      ],
      mesh=mesh,
      scratch_types=[
          pltpu.VMEM((GRP, CH), jnp.int32),
          pltpu.VMEM((GRP, CH), jnp.int32),
          pltpu.VMEM((CH, 128), jnp.float32),
          pltpu.VMEM_SHARED((AGR, 128), jnp.float32),
          pltpu.SemaphoreType.DMA,
      ],
  )
  def k(h_hbm, src_hbm, dst_hbm, z_hbm, out_a, out_b,
        src_v, dst_v, rows_v, sh, sem0):
    c = lax.axis_index("c")
    s = lax.axis_index("s")
    pltpu.sync_copy(z_hbm.at[pl.ds(s * STRIPE, STRIPE)],
                    sh.at[pl.ds(s * STRIPE, STRIPE)])
    plsc.subcore_barrier()
    wid = c * 16 + s

    def body(gg, carry):
      pltpu.sync_copy(src_hbm.at[gg], src_v)
      pltpu.sync_copy(dst_hbm.at[gg], dst_v)
      for q in range(GRP):
        pltpu.async_copy(h_hbm.at[src_v.at[q]], rows_v, sem0).wait()
        pltpu.sync_copy(rows_v, sh.at[dst_v.at[q]], add=True)
      return carry

    @pl.when(wid == 0)
    def _():
      lax.fori_loop(0, 32 * NG, body, 0)
    plsc.subcore_barrier()

    @pl.when(c == 0)
    def _():
      pltpu.sync_copy(sh.at[pl.ds(s * STRIPE, STRIPE)],
                      out_a.at[pl.ds(s * STRIPE, STRIPE)])

    @pl.when(c == 1)
    def _():
      pltpu.sync_copy(sh.at[pl.ds(s * STRIPE, STRIPE)],
                      out_b.at[pl.ds(s * STRIPE, STRIPE)])

  return k(h128, src3d, dst3d, zrows)


def _tc_embed(x, w, b2d):
  def body(x_ref, w_ref, b_ref, out_ref):
    h = jnp.dot(x_ref[...], w_ref[...],
                preferred_element_type=jnp.float32) + b_ref[...]
    out_ref[pl.ds(0, N), pl.ds(0, H)] = h
    out_ref[pl.ds(0, N), pl.ds(H, H)] = jnp.zeros((N, H), jnp.float32)
    out_ref[pl.ds(N, NP - N), :] = jnp.zeros((NP - N, 128), jnp.float32)

  return pl.pallas_call(
      body, out_shape=jax.ShapeDtypeStruct((NP, 128), jnp.float32))(x, w, b2d)


def _tc_layer(h128, p0a, p0b, p1a, p1b, w1, b1r, w2, b2r, gr, br):
  def body(h_ref, p0a_ref, p0b_ref, p1a_ref, p1b_ref, w1_ref, b1_ref, w2_ref,
           b2_ref, g_ref, be_ref, out_ref):
    agg_lo = p0a_ref[pl.ds(0, HALF), pl.ds(0, H)] + p0b_ref[pl.ds(0, HALF),
                                                            pl.ds(0, H)]
    agg_hi = p1a_ref[pl.ds(0, HALF), pl.ds(0, H)] + p1b_ref[pl.ds(0, HALF),
                                                            pl.ds(0, H)]
    agg = jnp.concatenate([agg_lo, agg_hi], axis=0)
    z = h_ref[:, pl.ds(0, H)] + agg
    z = jnp.maximum(
        jnp.dot(z, w1_ref[...], preferred_element_type=jnp.float32)
        + b1_ref[...], 0.0)
    z = jnp.dot(z, w2_ref[...], preferred_element_type=jnp.float32) + b2_ref[...]
    mask = lax.broadcasted_iota(jnp.int32, (NP, 1), 0) < N
    zm = jnp.where(mask, z, 0.0)
    mean = jnp.sum(zm, axis=0, keepdims=True) / N
    dev = jnp.where(mask, z - mean, 0.0)
    var = jnp.sum(dev * dev, axis=0, keepdims=True) / N
    zn = (z - mean) * lax.rsqrt(var + 1e-5) * g_ref[...] + be_ref[...]
    hout = jnp.where(mask, jnp.maximum(zn, 0.0), 0.0)
    out_ref[:, pl.ds(0, H)] = hout
    out_ref[:, pl.ds(H, H)] = jnp.zeros((NP, H), jnp.float32)

  return pl.pallas_call(
      body, out_shape=jax.ShapeDtypeStruct((NP, 128), jnp.float32))(
          h128, p0a, p0b, p1a, p1b, w1, b1r, w2, b2r, gr, br)


def _tc_head(h128, batch2d, w1, b1r, w2row, b2s):
  def body(h_ref, batch_ref, w1_ref, b1_ref, w2_ref, b2_ref, out_ref):
    h = h_ref[:, pl.ds(0, H)]
    gids = lax.broadcasted_iota(jnp.int32, (G, NP), 0)
    oh = (batch_ref[...] == gids).astype(jnp.float32)
    gf = jnp.dot(oh, h, preferred_element_type=jnp.float32)
    g1 = jnp.maximum(
        jnp.dot(gf, w1_ref[...], preferred_element_type=jnp.float32)
        + b1_ref[...], 0.0)
    out = jnp.sum(g1 * w2_ref[...], axis=1) + b2_ref[0, 0]
    out_ref[...] = out[None, :]

  return pl.pallas_call(
      body, out_shape=jax.ShapeDtypeStruct((1, G), jnp.float32))(
          h128, batch2d, w1, b1r, w2row, b2s)


def kernel(x, edge_index, batch, W_embed, b_embed, W1, b1, W2, b2, gamma, beta,
           W_fc1, b_fc1, W_fc2, b_fc2):
  src = edge_index[0].astype(jnp.int32)
  dst = edge_index[1].astype(jnp.int32)
  pad = E_PAD - E
  src_p = jnp.concatenate([src, jnp.full((pad,), N, jnp.int32)])
  dst_p = jnp.concatenate([dst, jnp.full((pad,), -1, jnp.int32)])
  src3d = src_p.reshape(32 * NG, GRP, CH)
  dst3d = []
  dummy_spread = DUMMY + (jnp.arange(E_PAD, dtype=jnp.int32) % (AGR - HALF))
  for half in range(2):
    lo = half * HALF
    inh = (dst_p >= lo) & (dst_p < lo + HALF)
    dst3d.append(
        jnp.where(inh, dst_p - lo, dummy_spread).reshape(32 * NG, GRP, CH))
  zrows = jnp.zeros((AGR, 128), jnp.float32)
  batch2d = jnp.concatenate(
      [batch.astype(jnp.int32), jnp.full((NP - N,), G, jnp.int32)]).reshape(1, NP)

  h = _tc_embed(x, W_embed, b_embed.reshape(1, H))
  for l in range(L):
    p0a, p0b = _sc_half(h, src3d, dst3d[0], zrows)
    p1a, p1b = _sc_half(h, src3d, dst3d[1], zrows)
    h = _tc_layer(h, p0a, p0b, p1a, p1b, W1[l], b1[l].reshape(1, H), W2[l],
                  b2[l].reshape(1, H), gamma[l].reshape(1, H),
                  beta[l].reshape(1, H))
  out = _tc_head(h, batch2d, W_fc1, b_fc1.reshape(1, H), W_fc2.reshape(1, H),
                 b_fc2.reshape(1, 1))
  return out.reshape(-1)
